# baseline (speedup 1.0000x reference)
import jax
import jax.numpy as jnp
from jax import lax
from jax.experimental import pallas as pl
from jax.experimental.pallas import tpu as pltpu
from jax.experimental.pallas import tpu_sc as plsc

N = 10000
E = 320000
D_IN = 128
D_E = 16
H = 64
HW = 128
L = 3
G = 64

NC = 2
NS = 16
NW = NC * NS
EPW = E // NW
B = 72
NCH = EPW // B
BT = EPW - NCH * B
N8 = 10240
RPT = N8 // NS
ZB = 64
NZC = RPT // ZB

BEB = 8000
NBE = E // BEB


def _sc_body(h_hbm, ea_hbm, src_hbm, dst_hbm, out_hbm,
             srcall, dstv0, dstv1, dxt, rowsv0, rowsv1, eav0, eav1, aggsh,
             gsem0, gsem1, esem0, esem1, dsem0, dsem1):
    cid = lax.axis_index("c")
    sid = lax.axis_index("s")
    wid = sid * NC + cid
    base = wid * EPW
    bufs = ((dstv0, rowsv0, eav0, gsem0, esem0, dsem0),
            (dstv1, rowsv1, eav1, gsem1, esem1, dsem1))

    def _zrow(r, c):
        for j in range(HW // 16):
            rowsv0[r, pl.ds(j * 16, 16)] = jnp.zeros((16,), jnp.float32)
        return c

    lax.fori_loop(0, ZB, _zrow, 0)
    for k in range(NZC):
        pltpu.sync_copy(rowsv0.at[pl.ds(0, ZB)],
                        aggsh.at[pl.ds(sid * RPT + k * ZB, ZB)])

    pltpu.sync_copy(src_hbm.at[pl.ds(base, EPW)], srcall)
    plsc.subcore_barrier()

    def _relu(rowsv, eav, nrows):
        def _row(r, c2):
            for j in range(H // 16):
                sl = pl.ds(j * 16, 16)
                rowsv[r, sl] = jnp.maximum(rowsv[r, sl] + eav[r, sl], 0.0)
            return c2

        lax.fori_loop(0, nrows, _row, 0, unroll=8)

    def _issue(g, p):
        dstv, rowsv, eav, gsem, esem, dsem = bufs[p]
        off = base + g * B
        pltpu.async_copy(h_hbm.at[srcall.at[pl.ds(g * B, B)]], rowsv, gsem)
        pltpu.async_copy(ea_hbm.at[pl.ds(off, B)], eav, esem)
        pltpu.async_copy(dst_hbm.at[pl.ds(off, B)], dstv, dsem)

    def _process(g, p):
        dstv, rowsv, eav, gsem, esem, dsem = bufs[p]
        off = base + g * B
        pltpu.make_async_copy(
            h_hbm.at[srcall.at[pl.ds(g * B, B)]], rowsv, gsem).wait()
        pltpu.make_async_copy(ea_hbm.at[pl.ds(off, B)], eav, esem).wait()
        pltpu.make_async_copy(dst_hbm.at[pl.ds(off, B)], dstv, dsem).wait()
        _relu(rowsv, eav, B)
        pltpu.sync_copy(rowsv, aggsh.at[dstv], add=True)

    _issue(0, 0)
    _issue(1, 1)

    def _pair(it, c):
        g0 = 2 * it
        _process(g0, 0)

        @pl.when(g0 + 2 < NCH)
        def _():
            _issue(g0 + 2, 0)

        _process(g0 + 1, 1)

        @pl.when(g0 + 3 < NCH)
        def _():
            _issue(g0 + 3, 1)

        return c

    lax.fori_loop(0, NCH // 2, _pair, 0)

    offt = base + NCH * B
    pltpu.async_copy(h_hbm.at[srcall.at[pl.ds(NCH * B, BT)]],
                     rowsv0.at[pl.ds(0, BT)], gsem0)
    pltpu.async_copy(ea_hbm.at[pl.ds(offt, BT)], eav0.at[pl.ds(0, BT)], esem0)
    pltpu.async_copy(dst_hbm.at[pl.ds(offt, BT)], dxt, dsem0)
    pltpu.make_async_copy(h_hbm.at[srcall.at[pl.ds(NCH * B, BT)]],
                          rowsv0.at[pl.ds(0, BT)], gsem0).wait()
    pltpu.make_async_copy(ea_hbm.at[pl.ds(offt, BT)],
                          eav0.at[pl.ds(0, BT)], esem0).wait()
    pltpu.make_async_copy(dst_hbm.at[pl.ds(offt, BT)], dxt, dsem0).wait()
    _relu(rowsv0, eav0, BT)
    pltpu.sync_copy(rowsv0.at[pl.ds(0, BT)], aggsh.at[dxt], add=True)

    plsc.subcore_barrier()
    sl = pl.ds(sid * RPT, RPT)
    pltpu.sync_copy(aggsh.at[sl], out_hbm.at[cid, sl])


_sc_layer = pl.kernel(
    _sc_body,
    out_type=jax.ShapeDtypeStruct((NC, N8, HW), jnp.float32),
    mesh=plsc.VectorSubcoreMesh(
        core_axis_name="c", subcore_axis_name="s",
        num_cores=NC, num_subcores=NS),
    scratch_types=[
        pltpu.VMEM((EPW,), jnp.int32),
        pltpu.VMEM((B,), jnp.int32),
        pltpu.VMEM((B,), jnp.int32),
        pltpu.VMEM((BT,), jnp.int32),
        pltpu.VMEM((B, HW), jnp.float32),
        pltpu.VMEM((B, HW), jnp.float32),
        pltpu.VMEM((B, H), jnp.float32),
        pltpu.VMEM((B, H), jnp.float32),
        pltpu.VMEM_SHARED((N8, HW), jnp.float32),
        pltpu.SemaphoreType.DMA,
        pltpu.SemaphoreType.DMA,
        pltpu.SemaphoreType.DMA,
        pltpu.SemaphoreType.DMA,
        pltpu.SemaphoreType.DMA,
        pltpu.SemaphoreType.DMA,
    ],
)


def _node_proj_body(x_ref, w_ref, b_ref, o_ref):
    o_ref[...] = (
        jnp.dot(x_ref[...], w_ref[...], preferred_element_type=jnp.float32)
        + b_ref[...]
    )


_node_proj = pl.pallas_call(
    _node_proj_body,
    out_shape=jax.ShapeDtypeStruct((N, HW), jnp.float32),
)

_edge_proj = pl.pallas_call(
    _node_proj_body,
    grid=(NBE,),
    in_specs=[
        pl.BlockSpec((BEB, D_E), lambda i: (i, 0)),
        pl.BlockSpec((D_E, H), lambda i: (0, 0)),
        pl.BlockSpec((1, H), lambda i: (0, 0)),
    ],
    out_specs=pl.BlockSpec((BEB, H), lambda i: (i, 0)),
    out_shape=jax.ShapeDtypeStruct((E, H), jnp.float32),
)


def _layer_core(h_ref, agg_ref, w1_ref, b1_ref, w2_ref, b2_ref, g_ref, b_ref):
    z = h_ref[:, :H] + agg_ref[0, :N, :H] + agg_ref[1, :N, :H]
    z = jnp.maximum(
        jnp.dot(z, w1_ref[...], preferred_element_type=jnp.float32) + b1_ref[...],
        0.0,
    )
    z = jnp.dot(z, w2_ref[...], preferred_element_type=jnp.float32) + b2_ref[...]
    m = jnp.mean(z, axis=0, keepdims=True)
    zc = z - m
    var = jnp.mean(zc * zc, axis=0, keepdims=True)
    inv = g_ref[...] * lax.rsqrt(var + 1e-5)
    return jnp.maximum(zc * inv + b_ref[...], 0.0)


def _layer_body(h_ref, agg_ref, w1_ref, b1_ref, w2_ref, b2_ref, g_ref, b_ref,
                o_ref):
    hb = _layer_core(h_ref, agg_ref, w1_ref, b1_ref, w2_ref, b2_ref,
                     g_ref, b_ref)
    o_ref[...] = jnp.concatenate(
        [hb, jnp.zeros((N, HW - H), jnp.float32)], axis=1)


_layer_tc = pl.pallas_call(
    _layer_body,
    out_shape=jax.ShapeDtypeStruct((N, HW), jnp.float32),
)


def _layer_pool_body(h_ref, agg_ref, w1_ref, b1_ref, w2_ref, b2_ref,
                     g_ref, b_ref, bat_ref, o_ref, emb_ref):
    hb = _layer_core(h_ref, agg_ref, w1_ref, b1_ref, w2_ref, b2_ref,
                     g_ref, b_ref)
    o_ref[...] = hb
    ids = bat_ref[0, :]
    onehot = (ids[:, None]
              == lax.broadcasted_iota(jnp.int32, (N, G), 1)).astype(jnp.float32)
    sums = lax.dot_general(
        onehot, hb, (((0,), (0,)), ((), ())), preferred_element_type=jnp.float32)
    cnt = jnp.sum(onehot, axis=0)[:, None]
    emb_ref[...] = sums / jnp.maximum(cnt, 1.0)


_layer_pool_tc = pl.pallas_call(
    _layer_pool_body,
    out_shape=[
        jax.ShapeDtypeStruct((N, H), jnp.float32),
        jax.ShapeDtypeStruct((G, H), jnp.float32),
    ],
)


def kernel(x, edge_attr, node_W, node_b, edge_W, edge_b,
           mlp_W1, mlp_b1, mlp_W2, mlp_b2, bn_g, bn_b, edge_index, batch):
    src = edge_index[0]
    dst = edge_index[1]
    batch2 = batch.reshape(1, N)
    node_Wp = jnp.pad(node_W, ((0, 0), (0, HW - H)))
    node_bp = jnp.pad(node_b, (0, HW - H)).reshape(1, HW)

    h = _node_proj(x, node_Wp, node_bp)
    ea = _edge_proj(edge_attr, edge_W, edge_b.reshape(1, H))

    emb = None
    for l in range(L):
        agg = _sc_layer(h, ea, src, dst)
        args = (h, agg, mlp_W1[l], mlp_b1[l].reshape(1, H),
                mlp_W2[l], mlp_b2[l].reshape(1, H),
                bn_g[l].reshape(1, H), bn_b[l].reshape(1, H))
        if l < L - 1:
            h = _layer_tc(*args)
        else:
            h, emb = _layer_pool_tc(*args, batch2)
    return (h, emb)
